# baseline (speedup 1.0000x reference)
import functools

import jax
import jax.numpy as jnp
from jax import lax
from jax.experimental import pallas as pl
from jax.experimental.pallas import tpu as pltpu
from jax.experimental.pallas import tpu_sc as plsc

B = 4096
M = 16
D = 64
N = M * M
NC = 2
NS = 16
NW = NC * NS
ROWS = B // NW
BLK = 1024
NCHUNK = 4
CROWS = ROWS // NCHUNK
CWORDS = CROWS * N


def _nt_dot(a, b):
    return lax.dot_general(
        a, b, (((1,), (1,)), ((), ())),
        preferred_element_type=jnp.float32,
        precision=lax.Precision.HIGHEST)


def _scores_body(x_ref, w_ref, s_ref):
    w3 = w_ref[...]
    wt = jnp.concatenate([w3[:, k, :] for k in range(M)], axis=0)
    ones = jnp.ones((8, D), jnp.float32)
    w2 = _nt_dot(ones, wt * wt)[0:1, :]
    s = w2 - 2.0 * _nt_dot(x_ref[...], wt)
    s_ref[:, 0, :, :] = s[:, :128].reshape(BLK // 8, 8, 128)
    s_ref[:, 1, :, :] = s[:, 128:].reshape(BLK // 8, 8, 128)


def _tc_scores(x, weights):
    return pl.pallas_call(
        _scores_body,
        grid=(B // BLK,),
        in_specs=[
            pl.BlockSpec((BLK, D), lambda i: (i, 0)),
            pl.BlockSpec((M, M, D), lambda i: (0, 0, 0)),
        ],
        out_specs=pl.BlockSpec((BLK // 8, 2, 8, 128), lambda i: (i, 0, 0, 0)),
        out_shape=jax.ShapeDtypeStruct((B // 8, 2, 8, 128), jnp.float32),
    )(x, weights)


def _argmin_body(s_hbm, o_hbm, s_v, o_v, sems):
    wid = lax.axis_index("s") * NC + lax.axis_index("c")
    base = wid * ROWS * N
    copies = [
        pltpu.async_copy(
            s_hbm.at[pl.ds(base + c * CWORDS, CWORDS)],
            s_v.at[pl.ds(c * CWORDS, CWORDS)],
            sems.at[c])
        for c in range(NCHUNK)
    ]

    def row(r, carry):
        rbase = (r >> 3) * 2048 + (r & 7) * 128
        best = s_v[pl.ds(rbase, M)]
        bidx = jnp.zeros((M,), jnp.int32)
        for k in range(1, M):
            off = (k >> 3) * 1024 + (k & 7) * 16
            v = s_v[pl.ds(rbase + off, M)]
            m = v < best
            best = jnp.where(m, v, best)
            bidx = jnp.where(m, jnp.int32(k), bidx)
        o_v[pl.ds(r * M, M)] = bidx
        return carry

    for c in range(NCHUNK):
        copies[c].wait()
        lax.fori_loop(c * CROWS, (c + 1) * CROWS, row, 0, unroll=2)
    pltpu.sync_copy(o_v, o_hbm.at[pl.ds(wid * ROWS * M, ROWS * M)])


@functools.cache
def _sc_argmin():
    return pl.kernel(
        _argmin_body,
        out_type=jax.ShapeDtypeStruct((B * M,), jnp.int32),
        mesh=plsc.VectorSubcoreMesh(core_axis_name="c", subcore_axis_name="s"),
        scratch_types=[
            pltpu.VMEM((ROWS * N,), jnp.float32),
            pltpu.VMEM((ROWS * M,), jnp.int32),
            pltpu.SemaphoreType.DMA((NCHUNK,)),
        ],
    )


def kernel(x, weights):
    scores = _tc_scores(x, weights)
    bmu = _sc_argmin()(scores.reshape(-1))
    return bmu.reshape(B, M)


# speedup vs baseline: 1.0455x; 1.0455x over previous
import functools

import jax
import jax.numpy as jnp
from jax import lax
from jax.experimental import pallas as pl
from jax.experimental.pallas import tpu as pltpu
from jax.experimental.pallas import tpu_sc as plsc

B = 4096
M = 16
D = 64
N = M * M
NC = 2
NS = 16
NW = NC * NS
ROWS = B // NW
HROWS = ROWS // 2
HWORDS = HROWS * N
BLK = 2048
PSTRIDE = (B // 16) * 2048


def _nt_dot(a, b):
    return lax.dot_general(
        a, b, (((1,), (1,)), ((), ())),
        preferred_element_type=jnp.float32,
        precision=lax.Precision.HIGHEST)


def _scores_body(x_ref, w_ref, s_ref):
    w3 = w_ref[...]
    wt = jnp.concatenate([w3[:, k, :] for k in range(M)], axis=0)
    ones = jnp.ones((8, D), jnp.float32)
    w2 = _nt_dot(ones, wt * wt)[0:1, :]
    xp = x_ref[...]
    for p in range(2):
        xh = xp[:, p * D:(p + 1) * D]
        s = w2 - 2.0 * _nt_dot(xh, wt)
        s_ref[p, :, 0, :, :] = s[:, :128].reshape(BLK // 16, 8, 128)
        s_ref[p, :, 1, :, :] = s[:, 128:].reshape(BLK // 16, 8, 128)


def _tc_scores(x, weights):
    return pl.pallas_call(
        _scores_body,
        grid=(B // BLK,),
        in_specs=[
            pl.BlockSpec((BLK // 2, 128), lambda i: (i, 0)),
            pl.BlockSpec((M, M, D), lambda i: (0, 0, 0)),
        ],
        out_specs=pl.BlockSpec(
            (2, BLK // 16, 2, 8, 128), lambda i: (0, i, 0, 0, 0)),
        out_shape=jax.ShapeDtypeStruct((2, B // 16, 2, 8, 128), jnp.float32),
    )(x.reshape(B // 2, 128), weights)


def _argmin_body(s_hbm, o_hbm, s_v, o_v, sems):
    wid = lax.axis_index("s") * NC + lax.axis_index("c")
    base = wid * ROWS
    qbase = (base // 2) * N
    copies = [
        pltpu.async_copy(
            s_hbm.at[pl.ds(p * PSTRIDE + qbase, HWORDS)],
            s_v.at[pl.ds(p * HWORDS, HWORDS)],
            sems.at[p])
        for p in range(2)
    ]

    def make_row(p):
        def row(r, carry):
            rbase = p * HWORDS + (r >> 3) * 2048 + (r & 7) * 128
            best = s_v[pl.ds(rbase, M)]
            bidx = jnp.zeros((M,), jnp.int32)
            for k in range(1, M):
                off = (k >> 3) * 1024 + (k & 7) * 16
                v = s_v[pl.ds(rbase + off, M)]
                m = v < best
                best = jnp.where(m, v, best)
                bidx = jnp.where(m, jnp.int32(k), bidx)
            o_v[pl.ds((2 * r + p) * M, M)] = bidx
            return carry
        return row

    for p in range(2):
        copies[p].wait()
        lax.fori_loop(0, HROWS, make_row(p), 0)
    pltpu.sync_copy(o_v, o_hbm.at[pl.ds(base * M, ROWS * M)])


@functools.cache
def _sc_argmin():
    return pl.kernel(
        _argmin_body,
        out_type=jax.ShapeDtypeStruct((B * M,), jnp.int32),
        mesh=plsc.VectorSubcoreMesh(core_axis_name="c", subcore_axis_name="s"),
        scratch_types=[
            pltpu.VMEM((ROWS * N,), jnp.float32),
            pltpu.VMEM((ROWS * M,), jnp.int32),
            pltpu.SemaphoreType.DMA((2,)),
        ],
    )


def kernel(x, weights):
    scores = _tc_scores(x, weights)
    bmu = _sc_argmin()(scores.reshape(-1))
    return bmu.reshape(B, M)


# speedup vs baseline: 1.0953x; 1.0476x over previous
import functools

import jax
import jax.numpy as jnp
from jax import lax
from jax.experimental import pallas as pl
from jax.experimental.pallas import tpu as pltpu
from jax.experimental.pallas import tpu_sc as plsc

B = 4096
M = 16
D = 64
N = M * M
NC = 2
NS = 16
NW = NC * NS
ROWS = B // NW
HROWS = ROWS // 2
HWORDS = HROWS * N
BLK = 2048
PSTRIDE = (B // 16) * 2048


def _nt_dot(a, b):
    return lax.dot_general(
        a, b, (((1,), (1,)), ((), ())),
        preferred_element_type=jnp.float32,
        precision=lax.Precision.HIGHEST)


def _scores_body(x_ref, w_ref, s_ref):
    w3 = w_ref[...]
    wt = jnp.concatenate([w3[:, k, :] for k in range(M)], axis=0)
    ones = jnp.ones((8, D), jnp.float32)
    w2 = _nt_dot(ones, wt * wt)[0:1, :]
    xp = x_ref[...]
    for p in range(2):
        xh = xp[:, p * D:(p + 1) * D]
        s = w2 - 2.0 * _nt_dot(xh, wt)
        s_ref[p, :, 0, :, :] = s[:, :128].reshape(BLK // 16, 8, 128)
        s_ref[p, :, 1, :, :] = s[:, 128:].reshape(BLK // 16, 8, 128)


def _tc_scores(x, weights):
    return pl.pallas_call(
        _scores_body,
        grid=(B // BLK,),
        in_specs=[
            pl.BlockSpec((BLK // 2, 128), lambda i: (i, 0)),
            pl.BlockSpec((M, M, D), lambda i: (0, 0, 0)),
        ],
        out_specs=pl.BlockSpec(
            (2, BLK // 16, 2, 8, 128), lambda i: (0, i, 0, 0, 0)),
        out_shape=jax.ShapeDtypeStruct((2, B // 16, 2, 8, 128), jnp.float32),
    )(x.reshape(B // 2, 128), weights)


def _argmin_body(s_hbm, o_hbm, s_v, o_v, sems):
    wid = lax.axis_index("s") * NC + lax.axis_index("c")
    base = wid * ROWS
    qbase = (base // 2) * N
    copies = [
        pltpu.async_copy(
            s_hbm.at[pl.ds(p * PSTRIDE + qbase, HWORDS)],
            s_v.at[pl.ds(p * HWORDS, HWORDS)],
            sems.at[p])
        for p in range(2)
    ]

    def make_row(p):
        def row(r):
            rbase = p * HWORDS + (r >> 3) * 2048 + (r & 7) * 128
            best = s_v[pl.ds(rbase, M)]
            bidx = jnp.zeros((M,), jnp.int32)
            for k in range(1, M):
                off = (k >> 3) * 1024 + (k & 7) * 16
                v = s_v[pl.ds(rbase + off, M)]
                m = v < best
                best = jnp.where(m, v, best)
                bidx = jnp.where(m, jnp.int32(k), bidx)
            o_v[pl.ds((2 * r + p) * M, M)] = bidx
        return row

    for p in range(2):
        copies[p].wait()
        plsc.parallel_loop(0, HROWS, unroll=2)(make_row(p))
    pltpu.sync_copy(o_v, o_hbm.at[pl.ds(base * M, ROWS * M)])


@functools.cache
def _sc_argmin():
    return pl.kernel(
        _argmin_body,
        out_type=jax.ShapeDtypeStruct((B * M,), jnp.int32),
        mesh=plsc.VectorSubcoreMesh(core_axis_name="c", subcore_axis_name="s"),
        scratch_types=[
            pltpu.VMEM((ROWS * N,), jnp.float32),
            pltpu.VMEM((ROWS * M,), jnp.int32),
            pltpu.SemaphoreType.DMA((2,)),
        ],
    )


def kernel(x, weights):
    scores = _tc_scores(x, weights)
    bmu = _sc_argmin()(scores.reshape(-1))
    return bmu.reshape(B, M)
